# Initial kernel scaffold; baseline (speedup 1.0000x reference)
import jax
import jax.numpy as jnp
from jax.experimental import pallas as pl


def kernel(signal, bc, W_templates, b_templates, W_dense, b_dense):
    raise NotImplementedError("write your pallas kernel here")


# speedup vs baseline: 5.2813x; 5.2813x over previous
import functools

import jax
import jax.numpy as jnp
from jax import lax
from jax.experimental import pallas as pl
from jax.experimental.pallas import tpu as pltpu
from jax.experimental.pallas import tpu_sc as plsc

N = 10000
NR = 5
NA = 8
RA = NR * NA
T = 128
NROT = 8
NCLS = 10

NTILES = 32
CHUNK = 314
GROUPS = CHUNK * RA // 16
BC_E = CHUNK * RA * 3 * 2


def _sc_interp_body(sig_hbm, bc_hbm, out_hbm, sig_v, bc_v, out_v):
    cid = lax.axis_index("c")
    sid = lax.axis_index("s")
    wid = sid * 2 + cid
    vstart = jnp.minimum(wid * CHUNK, N - CHUNK)
    pltpu.sync_copy(sig_hbm, sig_v)
    pltpu.sync_copy(bc_hbm.at[pl.ds(vstart * (RA * 6), BC_E)], bc_v)

    def body(g, carry):
        p = g * 16 + lax.iota(jnp.int32, 16)
        acc = jnp.zeros((16,), jnp.float32)
        for k in range(3):
            pos = (3 * p + k) * 2
            iv = plsc.load_gather(bc_v, [pos])
            wv = plsc.load_gather(bc_v, [pos + 1])
            sv = plsc.load_gather(sig_v, [iv.astype(jnp.int32)])
            acc = acc + sv * wv
        out_v[pl.ds(g * 16, 16)] = acc
        return carry

    lax.fori_loop(0, GROUPS, body, 0)
    pltpu.sync_copy(out_v, out_hbm.at[pl.ds(vstart * RA, CHUNK * RA)])


@jax.jit
def _sc_interp(sig, bc_flat):
    mesh = plsc.VectorSubcoreMesh(core_axis_name="c", subcore_axis_name="s")
    fn = functools.partial(
        pl.kernel,
        out_type=jax.ShapeDtypeStruct((N * RA,), jnp.float32),
        mesh=mesh,
        scratch_types=[
            pltpu.VMEM((N,), jnp.float32),
            pltpu.VMEM((BC_E,), jnp.float32),
            pltpu.VMEM((CHUNK * RA,), jnp.float32),
        ],
        compiler_params=pltpu.CompilerParams(needs_layout_passes=False),
    )(_sc_interp_body)
    return fn(sig, bc_flat)


BN = 400
NBLK = N // BN


def _tc_head_body(interp_ref, w2_ref, bt_ref, wd_ref, bd_ref, out_ref):
    i = pl.program_id(0)
    interp = interp_ref[...]
    w2 = w2_ref[...]
    bt = bt_ref[...]

    best_ss = None
    best_act = None
    for o in range(NROT):
        if o == 0:
            wr = w2
        else:
            wr = jnp.concatenate([w2[:, o:, :], w2[:, :o, :]], axis=1)
        wr = wr.reshape(RA, T)
        conv = lax.dot_general(interp, wr, (((1,), (0,)), ((), ())),
                               preferred_element_type=jnp.float32) + bt
        act = jnp.maximum(conv, 0.0)
        ss = jnp.sum(act * act, axis=1, keepdims=True)
        if o == 0:
            best_ss, best_act = ss, act
        else:
            m = ss > best_ss
            best_act = jnp.where(m, act, best_act)
            best_ss = jnp.where(m, ss, best_ss)

    t_row = lax.broadcasted_iota(jnp.int32, (T, T * NCLS), 0)
    t_col = lax.broadcasted_iota(jnp.int32, (T, T * NCLS), 1) // NCLS
    e2 = (t_row == t_col).astype(jnp.float32)
    p10 = lax.dot_general(best_act, e2, (((1,), (0,)), ((), ())),
                          preferred_element_type=jnp.float32)
    z = p10 * wd_ref[...]
    cols = jnp.sum(z, axis=0, keepdims=True)
    c_row = lax.broadcasted_iota(jnp.int32, (T * NCLS, NCLS), 0) % NCLS
    c_col = lax.broadcasted_iota(jnp.int32, (T * NCLS, NCLS), 1)
    c2 = (c_row == c_col).astype(jnp.float32)
    partial = lax.dot_general(cols, c2, (((1,), (0,)), ((), ())),
                              preferred_element_type=jnp.float32)

    @pl.when(i == 0)
    def _():
        out_ref[...] = bd_ref[...]

    out_ref[...] += partial


@jax.jit
def _tc_head(interp, w2, bt, wd2, bd):
    return pl.pallas_call(
        _tc_head_body,
        grid=(NBLK,),
        in_specs=[
            pl.BlockSpec((BN, RA), lambda i: (i, 0)),
            pl.BlockSpec((NR, NA, T), lambda i: (0, 0, 0)),
            pl.BlockSpec((1, T), lambda i: (0, 0)),
            pl.BlockSpec((BN, T * NCLS), lambda i: (i, 0)),
            pl.BlockSpec((1, NCLS), lambda i: (0, 0)),
        ],
        out_specs=pl.BlockSpec((1, NCLS), lambda i: (0, 0)),
        out_shape=jax.ShapeDtypeStruct((1, NCLS), jnp.float32),
    )(interp, w2, bt, wd2, bd)


def kernel(signal, bc, W_templates, b_templates, W_dense, b_dense):
    sig = signal[0, :, 0]
    bc_flat = bc.reshape(-1)
    interp_flat = _sc_interp(sig, bc_flat)
    interp = interp_flat.reshape(N, RA)
    w2 = jnp.transpose(W_templates[:, :, :, 0], (1, 2, 0))
    wd2 = W_dense.reshape(N, T * NCLS)
    return _tc_head(interp, w2, b_templates.reshape(1, T),
                    wd2, b_dense.reshape(1, NCLS))


# speedup vs baseline: 100.8135x; 19.0889x over previous
import functools

import jax
import jax.numpy as jnp
from jax import lax
from jax.experimental import pallas as pl
from jax.experimental.pallas import tpu as pltpu
from jax.experimental.pallas import tpu_sc as plsc

N = 10000
NR = 5
NA = 8
RA = NR * NA
T = 128
NROT = 8
NCLS = 10

NTILES = 32
NCH = 4
CW = 2560
NGRP = CW // 16
ITEMS_PER_TILE = RA * NCH // NTILES
NP = 10240


def _sc_interp_body(sig_hbm, bc_hbm, out_hbm, sig_v, bc_v, out_v, sem):
    cid = lax.axis_index("c")
    sid = lax.axis_index("s")
    wid = sid * 2 + cid
    pltpu.sync_copy(sig_hbm, sig_v)

    def item_body(j, carry):
        item = wid * ITEMS_PER_TILE + j
        ra = item // NCH
        ch = item - ra * NCH
        n0 = jnp.minimum(ch * CW, N - CW)
        base = ra * (6 * N) + n0
        for q in range(6):
            pltpu.make_async_copy(
                bc_hbm.at[pl.ds(base + q * N, CW)],
                bc_v.at[pl.ds(q * CW, CW)], sem).start()
        pltpu.make_async_copy(bc_hbm.at[pl.ds(0, 6 * CW)], bc_v, sem).wait()

        def grp(g, c2):
            acc = jnp.zeros((16,), jnp.float32)
            for k in range(3):
                iv = bc_v[pl.ds(2 * k * CW + g * 16, 16)]
                wv = bc_v[pl.ds((2 * k + 1) * CW + g * 16, 16)]
                sv = plsc.load_gather(sig_v, [iv.astype(jnp.int32)])
                acc = acc + sv * wv
            out_v[pl.ds(g * 16, 16)] = acc
            return c2

        lax.fori_loop(0, NGRP, grp, 0)
        pltpu.sync_copy(out_v, out_hbm.at[pl.ds(ra * NP + n0, CW)])
        return carry

    lax.fori_loop(0, ITEMS_PER_TILE, item_body, 0)


@jax.jit
def _sc_interp(sig, bc2f):
    mesh = plsc.VectorSubcoreMesh(core_axis_name="c", subcore_axis_name="s")
    fn = functools.partial(
        pl.kernel,
        out_type=jax.ShapeDtypeStruct((RA * NP,), jnp.float32),
        mesh=mesh,
        scratch_types=[
            pltpu.VMEM((N,), jnp.float32),
            pltpu.VMEM((6 * CW,), jnp.float32),
            pltpu.VMEM((CW,), jnp.float32),
            pltpu.SemaphoreType.DMA,
        ],
        compiler_params=pltpu.CompilerParams(needs_layout_passes=False),
    )(_sc_interp_body)
    return fn(sig, bc2f)


BN = 400
NBLK = N // BN
BV = 2048


def _tc_conv_body(interp_ref, w2_ref, bt_ref, out_ref):
    jt = interp_ref[...]
    w2 = w2_ref[...]
    bt = bt_ref[...]

    best_ss = None
    best_act = None
    for o in range(NROT):
        if o == 0:
            wr = w2
        else:
            wr = jnp.concatenate([w2[:, o:, :], w2[:, :o, :]], axis=1)
        wr = wr.reshape(RA, T)
        conv = lax.dot_general(jt, wr, (((0,), (0,)), ((), ())),
                               preferred_element_type=jnp.float32) + bt
        act = jnp.maximum(conv, 0.0)
        ss = jnp.sum(act * act, axis=1, keepdims=True)
        if o == 0:
            best_ss, best_act = ss, act
        else:
            m = ss > best_ss
            best_act = jnp.where(m, act, best_act)
            best_ss = jnp.where(m, ss, best_ss)
    out_ref[...] = best_act


@jax.jit
def _tc_conv(interp_t, w2, bt):
    return pl.pallas_call(
        _tc_conv_body,
        grid=(NP // BV,),
        in_specs=[
            pl.BlockSpec((RA, BV), lambda i: (0, i)),
            pl.BlockSpec((NR, NA, T), lambda i: (0, 0, 0)),
            pl.BlockSpec((1, T), lambda i: (0, 0)),
        ],
        out_specs=pl.BlockSpec((BV, T), lambda i: (i, 0)),
        out_shape=jax.ShapeDtypeStruct((NP, T), jnp.float32),
    )(interp_t, w2, bt)


KB = BN * T


def _tc_dense_body(q_ref, wdt_ref, bd_ref, out_ref):
    i = pl.program_id(0)
    q = q_ref[...].reshape(1, KB)
    part = jnp.sum(wdt_ref[...] * q, axis=1, keepdims=True)

    @pl.when(i == 0)
    def _():
        out_ref[...] = bd_ref[...]

    out_ref[...] += part


@jax.jit
def _tc_dense(q25, wdt, bd):
    return pl.pallas_call(
        _tc_dense_body,
        grid=(NBLK,),
        in_specs=[
            pl.BlockSpec((1, 1, KB), lambda i: (i, 0, 0)),
            pl.BlockSpec((NCLS, KB), lambda i: (0, i)),
            pl.BlockSpec((NCLS, 1), lambda i: (0, 0)),
        ],
        out_specs=pl.BlockSpec((NCLS, 1), lambda i: (0, 0)),
        out_shape=jax.ShapeDtypeStruct((NCLS, 1), jnp.float32),
    )(q25, wdt, bd)


def kernel(signal, bc, W_templates, b_templates, W_dense, b_dense):
    sig = signal[0, :, 0]
    bc2f = jnp.transpose(bc, (0, 2, 3, 4, 5, 1)).reshape(RA * 6 * N)
    interp_t = _sc_interp(sig, bc2f).reshape(RA, NP)
    w2 = jnp.transpose(W_templates[:, :, :, 0], (1, 2, 0))
    pooled = _tc_conv(interp_t, w2, b_templates.reshape(1, T))
    q25 = pooled[:N].reshape(NBLK, 1, KB)
    wdt = W_dense.T
    out = _tc_dense(q25, wdt, b_dense.reshape(NCLS, 1))
    return out.T


# speedup vs baseline: 109.7876x; 1.0890x over previous
import functools

import jax
import jax.numpy as jnp
from jax import lax
from jax.experimental import pallas as pl
from jax.experimental.pallas import tpu as pltpu
from jax.experimental.pallas import tpu_sc as plsc

N = 10000
NR = 5
NA = 8
RA = NR * NA
T = 128
NROT = 8
NCLS = 10

NTILES = 32
NCH = 4
CW = 2560
NGRP = CW // 16
ITEMS_PER_TILE = RA * NCH // NTILES
NP = 10240


def _sc_interp_body(sig_hbm, bc_hbm, out_hbm, sig_v, bc_v, out_v, sem):
    cid = lax.axis_index("c")
    sid = lax.axis_index("s")
    wid = sid * 2 + cid
    pltpu.sync_copy(sig_hbm, sig_v)

    def item_body(j, carry):
        item = wid * ITEMS_PER_TILE + j
        ra = item // NCH
        ch = item - ra * NCH
        n0 = jnp.minimum(ch * CW, N - CW)
        base = ra * (6 * N) + n0
        for q in range(6):
            pltpu.make_async_copy(
                bc_hbm.at[pl.ds(base + q * N, CW)],
                bc_v.at[pl.ds(q * CW, CW)], sem).start()
        pltpu.make_async_copy(bc_hbm.at[pl.ds(0, 6 * CW)], bc_v, sem).wait()

        def grp(g, c2):
            acc = jnp.zeros((16,), jnp.float32)
            for k in range(3):
                iv = bc_v[pl.ds(2 * k * CW + g * 16, 16)]
                wv = bc_v[pl.ds((2 * k + 1) * CW + g * 16, 16)]
                sv = plsc.load_gather(sig_v, [iv.astype(jnp.int32)])
                acc = acc + sv * wv
            out_v[pl.ds(g * 16, 16)] = acc
            return c2

        lax.fori_loop(0, NGRP, grp, 0)
        pltpu.sync_copy(out_v, out_hbm.at[pl.ds(ra * NP + n0, CW)])
        return carry

    lax.fori_loop(0, ITEMS_PER_TILE, item_body, 0)


@jax.jit
def _sc_interp(sig, bc2f):
    mesh = plsc.VectorSubcoreMesh(core_axis_name="c", subcore_axis_name="s")
    fn = functools.partial(
        pl.kernel,
        out_type=jax.ShapeDtypeStruct((RA * NP,), jnp.float32),
        mesh=mesh,
        scratch_types=[
            pltpu.VMEM((N,), jnp.float32),
            pltpu.VMEM((6 * CW,), jnp.float32),
            pltpu.VMEM((CW,), jnp.float32),
            pltpu.SemaphoreType.DMA,
        ],
        compiler_params=pltpu.CompilerParams(needs_layout_passes=False),
    )(_sc_interp_body)
    return fn(sig, bc2f)


BN = 1000
NBLK = N // BN
BV = 2048


def _tc_conv_body(interp_ref, w2_ref, bt_ref, out_ref):
    jt = interp_ref[...]
    w2 = w2_ref[...]
    btc = bt_ref[...]

    best_ss = None
    best_act = None
    for o in range(NROT):
        if o == 0:
            wr = w2
        else:
            wr = jnp.concatenate([w2[:, o:, :], w2[:, :o, :]], axis=1)
        wrt = jnp.transpose(wr.reshape(RA, T), (1, 0))
        conv = lax.dot_general(wrt, jt, (((1,), (0,)), ((), ())),
                               preferred_element_type=jnp.float32) + btc
        act = jnp.maximum(conv, 0.0)
        ss = jnp.sum(act * act, axis=0, keepdims=True)
        if o == 0:
            best_ss, best_act = ss, act
        else:
            m = ss > best_ss
            best_act = jnp.where(m, act, best_act)
            best_ss = jnp.where(m, ss, best_ss)
    out_ref[...] = jnp.transpose(best_act, (1, 0))


@jax.jit
def _tc_conv(interp_t, w2, bt):
    return pl.pallas_call(
        _tc_conv_body,
        grid=(NP // BV,),
        in_specs=[
            pl.BlockSpec((RA, BV), lambda i: (0, i)),
            pl.BlockSpec((NR, NA, T), lambda i: (0, 0, 0)),
            pl.BlockSpec((T, 1), lambda i: (0, 0)),
        ],
        out_specs=pl.BlockSpec((BV, T), lambda i: (i, 0)),
        out_shape=jax.ShapeDtypeStruct((NP, T), jnp.float32),
    )(interp_t, w2, bt)


KB = BN * T


def _tc_dense_body(q_ref, wdt_ref, bd_ref, out_ref):
    i = pl.program_id(0)
    q = q_ref[...].reshape(1, KB)
    part = jnp.sum(wdt_ref[...] * q, axis=1, keepdims=True)

    @pl.when(i == 0)
    def _():
        out_ref[...] = bd_ref[...]

    out_ref[...] += part


@jax.jit
def _tc_dense(q25, wdt, bd):
    return pl.pallas_call(
        _tc_dense_body,
        grid=(NBLK,),
        in_specs=[
            pl.BlockSpec((1, 1, KB), lambda i: (i, 0, 0)),
            pl.BlockSpec((NCLS, KB), lambda i: (0, i)),
            pl.BlockSpec((NCLS, 1), lambda i: (0, 0)),
        ],
        out_specs=pl.BlockSpec((NCLS, 1), lambda i: (0, 0)),
        out_shape=jax.ShapeDtypeStruct((NCLS, 1), jnp.float32),
    )(q25, wdt, bd)


def kernel(signal, bc, W_templates, b_templates, W_dense, b_dense):
    sig = signal[0, :, 0]
    bc2f = jnp.transpose(bc, (0, 2, 3, 4, 5, 1)).reshape(RA * 6 * N)
    interp_t = _sc_interp(sig, bc2f).reshape(RA, NP)
    w2 = jnp.transpose(W_templates[:, :, :, 0], (1, 2, 0))
    pooled = _tc_conv(interp_t, w2, b_templates.reshape(T, 1))
    q25 = pooled[:N].reshape(NBLK, 1, KB)
    wdt = W_dense.T
    out = _tc_dense(q25, wdt, b_dense.reshape(NCLS, 1))
    return out.T
